# Initial kernel scaffold; baseline (speedup 1.0000x reference)
import jax
import jax.numpy as jnp
from jax.experimental import pallas as pl


def kernel(x, edge_index, W1, b1, W2, b2, Wm1, bm1, Wm2, bm2, Wd1, bd1, Wd2, bd2, Wp, bp):
    raise NotImplementedError("write your pallas kernel here")


# speedup vs baseline: 10.5449x; 10.5449x over previous
import functools

import jax
import jax.numpy as jnp
from jax import lax
from jax.experimental import pallas as pl
from jax.experimental.pallas import tpu as pltpu
from jax.experimental.pallas import tpu_sc as plsc

N = 10000
D = 128
NHID = 32
LAT = 16
NCLS = 40
E = 320000

NTILES = 16
NCORES = 2
CHUNK = 128
SB = 10240
EPAD = 327680
NPAD = 10240

_SC_PARAMS = pltpu.CompilerParams(needs_layout_passes=False,
                                  use_tc_tiling_on_sc=False)


def _make_agg(width):
    G = width // 8
    GH = G // 2
    TPG = NTILES // GH
    EPT = EPAD // TPG
    NSB = EPT // SB
    RW = NPAD * 8 // TPG
    mesh = plsc.VectorSubcoreMesh(core_axis_name="c", subcore_axis_name="s")

    @functools.partial(
        pl.kernel,
        mesh=mesh,
        out_type=[jax.ShapeDtypeStruct((G * NPAD * 8,), jnp.float32),
                  jax.ShapeDtypeStruct((NCORES * NTILES, NPAD * 8),
                                       jnp.float32)],
        compiler_params=_SC_PARAMS,
        scratch_types=[
            pltpu.VMEM((SB // CHUNK, CHUNK), jnp.int32),
            pltpu.VMEM((SB // 16, 16), jnp.int32),
            pltpu.VMEM((CHUNK, 8), jnp.float32),
            pltpu.VMEM((CHUNK, 8), jnp.float32),
            pltpu.VMEM((NPAD * 8,), jnp.float32),
            pltpu.SemaphoreType.DMA,
            pltpu.SemaphoreType.DMA,
        ],
    )
    def agg(tab_hbm, srcg_hbm, dst_hbm, zeros_hbm, out_hbm, slots_hbm,
            src_v, dst_v, gb0, gb1, acc_v, semA, semB):
        c = lax.axis_index("c")
        s = lax.axis_index("s")
        gl = s // TPG
        rank = s % TPG
        g = c * GH + gl
        iota = lax.iota(jnp.int32, 16)
        half = iota >> 3
        colp = iota & 7

        pltpu.sync_copy(zeros_hbm, acc_v)

        def compute_chunk(gb, jrow):
            for j2 in range(8):
                dvec = dst_v[jrow * 8 + j2]
                for st in range(8):
                    pat = half + (2 * st)
                    dtak = lax.gather(
                        dvec, pat[:, None],
                        lax.GatherDimensionNumbers(
                            offset_dims=(), collapsed_slice_dims=(0,),
                            start_index_map=(0,)),
                        (1,), mode=lax.GatherScatterMode.PROMISE_IN_BOUNDS)
                    gval = plsc.load_gather(gb, [16 * j2 + 2 * st + half, colp])
                    plsc.addupdate_scatter(acc_v, [dtak * 8 + colp], gval)

        for sb in range(NSB):
            ebase = rank * EPT + sb * SB
            pltpu.sync_copy(srcg_hbm.at[g, pl.ds(ebase // CHUNK, SB // CHUNK)],
                            src_v)
            pltpu.sync_copy(dst_hbm.at[pl.ds(ebase // 16, SB // 16)], dst_v)
            pltpu.async_copy(tab_hbm.at[src_v.at[0]], gb0, semA)

            @pl.loop(0, SB // CHUNK // 2)
            def _(p):
                j = 2 * p
                nxt1 = jnp.minimum(j + 1, SB // CHUNK - 1)
                nxt2 = jnp.minimum(j + 2, SB // CHUNK - 1)
                pltpu.make_async_copy(tab_hbm.at[src_v.at[j]], gb0, semA).wait()
                pltpu.async_copy(tab_hbm.at[src_v.at[nxt1]], gb1, semB)
                compute_chunk(gb0, j)
                pltpu.make_async_copy(tab_hbm.at[src_v.at[nxt1]], gb1, semB).wait()
                pltpu.async_copy(tab_hbm.at[src_v.at[nxt2]], gb0, semA)
                compute_chunk(gb1, j + 1)

            pltpu.make_async_copy(tab_hbm.at[src_v.at[0]], gb0, semA).wait()

        pltpu.sync_copy(acc_v, slots_hbm.at[c * NTILES + s])
        plsc.subcore_barrier()
        roff = rank * RW
        for j in range(TPG):
            pltpu.sync_copy(
                slots_hbm.at[c * NTILES + gl * TPG + j, pl.ds(roff, RW)],
                acc_v.at[pl.ds(j * RW, RW)])

        @pl.loop(0, RW // 16)
        def _(i):
            v = acc_v[pl.ds(i * 16, 16)]
            for j in range(1, TPG):
                v = v + acc_v[pl.ds(j * RW + i * 16, 16)]
            acc_v[pl.ds(i * 16, 16)] = v

        pltpu.sync_copy(acc_v.at[pl.ds(0, RW)],
                        out_hbm.at[pl.ds(g * NPAD * 8 + rank * RW, RW)])

    return agg


def _make_deg():
    EPT = EPAD // (NCORES * NTILES)
    mesh = plsc.VectorSubcoreMesh(core_axis_name="c", subcore_axis_name="s")

    @functools.partial(
        pl.kernel,
        mesh=mesh,
        out_type=[jax.ShapeDtypeStruct((NCORES * NPAD * 8,), jnp.float32),
                  jax.ShapeDtypeStruct((NCORES * NTILES, NPAD), jnp.float32)],
        compiler_params=_SC_PARAMS,
        scratch_types=[
            pltpu.VMEM((EPT // 16, 16), jnp.int32),
            pltpu.VMEM((NPAD,), jnp.float32),
            pltpu.VMEM((NPAD,), jnp.float32),
            pltpu.VMEM((NPAD // NTILES * 8,), jnp.float32),
        ],
    )
    def deg(dst_hbm, zeros_hbm, out_hbm, slots_hbm, dst_v, acc_v, red_v, rep_v):
        c = lax.axis_index("c")
        s = lax.axis_index("s")
        wid = c * NTILES + s
        iota = lax.iota(jnp.int32, 16)
        ones16 = jnp.ones((16,), jnp.float32)
        pltpu.sync_copy(zeros_hbm.at[pl.ds(0, NPAD)], acc_v)
        pltpu.sync_copy(dst_hbm.at[pl.ds(wid * (EPT // 16), EPT // 16)], dst_v)

        @pl.loop(0, EPT // 16)
        def _(i):
            plsc.addupdate_scatter(acc_v, [dst_v[i]], ones16)

        pltpu.sync_copy(acc_v, slots_hbm.at[c * NTILES + s])
        plsc.subcore_barrier()
        zr = NPAD // NTILES
        for j in range(NTILES):
            pltpu.sync_copy(slots_hbm.at[c * NTILES + j, pl.ds(s * zr, zr)],
                            red_v.at[pl.ds(j * zr, zr)])

        @pl.loop(0, zr // 16)
        def _(i):
            v = red_v[pl.ds(i * 16, 16)]
            for j in range(1, NTILES):
                v = v + red_v[pl.ds(j * zr + i * 16, 16)]
            nvec = iota + i * 16
            for k in range(8):
                plsc.store_scatter(rep_v, [nvec * 8 + k], v)

        pltpu.sync_copy(rep_v,
                        out_hbm.at[pl.ds((c * NPAD + s * zr) * 8, zr * 8)])

    return deg


_BR = 1000


def _full(shape):
    return pl.BlockSpec(shape, lambda *g: (0,) * len(shape))


def _rows(shape_minor):
    return pl.BlockSpec((_BR,) + shape_minor,
                        lambda *g: (g[0],) + (0,) * len(shape_minor))


def _tc_dense(x, W1, Wm1, bm1, Wm2, bm2, Wd1, bd1, Wd2, bd2):
    def body(x_ref, w1_r, wm1_r, bm1_r, wm2_r, bm2_r, wd1_r, bd1_r, wd2_r,
             bd2_r, xw1_ref, zx_ref, fr_ref):
        xb = x_ref[...]
        xw1_ref[...] = jnp.dot(xb, w1_r[...], preferred_element_type=jnp.float32)
        h1 = jax.nn.relu(jnp.dot(xb, wm1_r[...],
                                 preferred_element_type=jnp.float32) + bm1_r[...])
        zx = jax.nn.relu(jnp.dot(h1, wm2_r[...],
                                 preferred_element_type=jnp.float32) + bm2_r[...])
        zx_ref[...] = zx
        f1 = jax.nn.relu(jnp.dot(zx, wd1_r[...],
                                 preferred_element_type=jnp.float32) + bd1_r[...])
        fr_ref[...] = jax.nn.sigmoid(
            jax.nn.relu(jnp.dot(f1, wd2_r[...],
                                preferred_element_type=jnp.float32) + bd2_r[...]))

    return pl.pallas_call(
        body,
        grid=(N // _BR,),
        in_specs=[
            _rows((D,)),
            _full((D, NHID)), _full((D, NHID)), _full((1, NHID)),
            _full((NHID, LAT)), _full((1, LAT)),
            _full((LAT, NHID)), _full((1, NHID)),
            _full((NHID, D)), _full((1, D)),
        ],
        out_specs=[_rows((NHID,)), _rows((LAT,)), _rows((D,))],
        out_shape=[
            jax.ShapeDtypeStruct((N, NHID), jnp.float32),
            jax.ShapeDtypeStruct((N, LAT), jnp.float32),
            jax.ShapeDtypeStruct((N, D), jnp.float32),
        ],
    )(x, W1, Wm1, bm1, Wm2, bm2, Wd1, bd1, Wd2, bd2)


def _tc_dinv_y1(degp0, degp1, xw1):
    def body(d0_ref, d1_ref, xw1_ref, dinv_ref, y1s_ref):
        gidx = pl.program_id(1)
        deg = d0_ref[...] + d1_ref[...] + 1.0
        dinv = lax.rsqrt(deg)
        dinv_ref[...] = dinv
        xw = xw1_ref[...]
        part = xw[:, 0:8]
        for gg in range(1, 4):
            part = jnp.where(gidx == gg, xw[:, 8 * gg:8 * gg + 8], part)
        y1s_ref[...] = dinv[:, :1] * part

    return pl.pallas_call(
        body,
        grid=(N // _BR, 4),
        in_specs=[
            pl.BlockSpec((_BR, 8), lambda i, g: (i, 0)),
            pl.BlockSpec((_BR, 8), lambda i, g: (i, 0)),
            pl.BlockSpec((_BR, NHID), lambda i, g: (i, 0)),
        ],
        out_specs=[
            pl.BlockSpec((_BR, 8), lambda i, g: (i, 0)),
            pl.BlockSpec((_BR, 8), lambda i, g: (g * (N // _BR) + i, 0)),
        ],
        out_shape=[
            jax.ShapeDtypeStruct((N, 8), jnp.float32),
            jax.ShapeDtypeStruct((4 * N, 8), jnp.float32),
        ],
    )(degp0, degp1, xw1)


def _tc_layer2_in(p1s, y1s, dinv, b1, W2):
    def body(p0, p1, p2, p3, y0, y1r, y2r, y3r, dinv_ref, b1_r, w2_r, y2s_ref):
        gidx = pl.program_id(1)
        dv = dinv_ref[...][:, :1]
        pcat = jnp.concatenate([p0[...], p1[...], p2[...], p3[...]], axis=1)
        ycat = jnp.concatenate([y0[...], y1r[...], y2r[...], y3r[...]], axis=1)
        h = jax.nn.relu(dv * (pcat + ycat) + b1_r[...])
        y2 = dv * jnp.dot(h, w2_r[...], preferred_element_type=jnp.float32)
        y2s_ref[...] = jnp.where(gidx == 0, y2[:, 0:8], y2[:, 8:16])

    def rows8():
        return pl.BlockSpec((_BR, 8), lambda i, g: (i, 0))

    return pl.pallas_call(
        body,
        grid=(N // _BR, 2),
        in_specs=[rows8() for _ in range(8)] + [
            rows8(),
            pl.BlockSpec((1, NHID), lambda i, g: (0, 0)),
            pl.BlockSpec((NHID, LAT), lambda i, g: (0, 0)),
        ],
        out_specs=[pl.BlockSpec((_BR, 8), lambda i, g: (g * (N // _BR) + i, 0))],
        out_shape=[jax.ShapeDtypeStruct((2 * N, 8), jnp.float32)],
    )(*p1s, *y1s, dinv, b1, W2)[0]


def _tc_head(p2s, y2s, dinv, b2, zx, Wp, bp):
    def body(p0, p1, y0, y1r, dinv_ref, b2_r, zx_ref, wp_r, bp_r,
             z_ref, pred_ref):
        dv = dinv_ref[...][:, :1]
        pcat = jnp.concatenate([p0[...], p1[...]], axis=1)
        ycat = jnp.concatenate([y0[...], y1r[...]], axis=1)
        za = jax.nn.relu(dv * (pcat + ycat) + b2_r[...])
        z = jnp.concatenate([za, zx_ref[...]], axis=1)
        z_ref[...] = z
        logits = jnp.dot(z, wp_r[...], preferred_element_type=jnp.float32) \
            + bp_r[...]
        pred_ref[...] = jax.nn.softmax(logits, axis=1)

    def rows8():
        return pl.BlockSpec((_BR, 8), lambda i: (i, 0))

    return pl.pallas_call(
        body,
        grid=(N // _BR,),
        in_specs=[rows8(), rows8(), rows8(), rows8(), rows8(),
                  _full((1, LAT)),
                  pl.BlockSpec((_BR, LAT), lambda i: (i, 0)),
                  _full((2 * LAT, NCLS)), _full((1, NCLS))],
        out_specs=[pl.BlockSpec((_BR, 2 * LAT), lambda i: (i, 0)),
                   pl.BlockSpec((_BR, NCLS), lambda i: (i, 0))],
        out_shape=[
            jax.ShapeDtypeStruct((N, 2 * LAT), jnp.float32),
            jax.ShapeDtypeStruct((N, NCLS), jnp.float32),
        ],
    )(*p2s, *y2s, dinv, b2, zx, Wp, bp)


_BI = 512
_BJ = 2048


def _tc_adj(z):
    def body(zi_ref, zj_ref, out_ref):
        out_ref[...] = jax.nn.sigmoid(
            lax.dot_general(zi_ref[...], zj_ref[...],
                            (((1,), (1,)), ((), ())),
                            preferred_element_type=jnp.float32))

    return pl.pallas_call(
        body,
        grid=(pl.cdiv(N, _BI), pl.cdiv(N, _BJ)),
        in_specs=[
            pl.BlockSpec((_BI, 2 * LAT), lambda i, j: (i, 0)),
            pl.BlockSpec((_BJ, 2 * LAT), lambda i, j: (j, 0)),
        ],
        out_specs=pl.BlockSpec((_BI, _BJ), lambda i, j: (i, j)),
        out_shape=jax.ShapeDtypeStruct((N, N), jnp.float32),
    )(z, z)


def kernel(x, edge_index, W1, b1, W2, b2, Wm1, bm1, Wm2, bm2,
           Wd1, bd1, Wd2, bd2, Wp, bp):
    pe = EPAD - E
    pad_src = jnp.arange(pe, dtype=jnp.int32) % N
    pad_dst = N + jnp.arange(pe, dtype=jnp.int32) % (NPAD - N)
    src_flat = jnp.concatenate([edge_index[0], pad_src])
    dst_flat = jnp.concatenate([edge_index[1], pad_dst])
    srcp = src_flat.reshape(EPAD // CHUNK, CHUNK)
    srcg = srcp[None, :, :] + (jnp.arange(4, dtype=jnp.int32) * N)[:, None, None]
    dstp16 = dst_flat.reshape(EPAD // 16, 16)
    zeros_f = jnp.zeros((NPAD * 8,), jnp.float32)
    b1r = b1.reshape(1, NHID)
    b2r = b2.reshape(1, LAT)
    bm1r = bm1.reshape(1, NHID)
    bm2r = bm2.reshape(1, LAT)
    bd1r = bd1.reshape(1, NHID)
    bd2r = bd2.reshape(1, D)
    bpr = bp.reshape(1, NCLS)

    _deg = _make_deg()
    _agg32 = _make_agg(NHID)
    _agg16 = _make_agg(LAT)

    deg_flat, _ = _deg(dstp16, zeros_f)
    degr = deg_flat.reshape(NCORES, NPAD, 8)
    xw1, zx, feat_recon = _tc_dense(x, W1, Wm1, bm1r, Wm2, bm2r,
                                    Wd1, bd1r, Wd2, bd2r)
    dinv, y1s = _tc_dinv_y1(degr[0][:N], degr[1][:N], xw1)

    p1_flat, _ = _agg32(y1s, srcg, dstp16, zeros_f)
    p1r = p1_flat.reshape(4, NPAD, 8)
    p1_parts = [p1r[g][:N] for g in range(4)]
    y1_parts = [y1s[g * N:(g + 1) * N] for g in range(4)]
    y2s = _tc_layer2_in(p1_parts, y1_parts, dinv, b1r, W2)

    p2_flat, _ = _agg16(y2s, srcg, dstp16, zeros_f)
    p2r = p2_flat.reshape(2, NPAD, 8)
    p2_parts = [p2r[g][:N] for g in range(2)]
    y2_parts = [y2s[g * N:(g + 1) * N] for g in range(2)]
    z, pred = _tc_head(p2_parts, y2_parts, dinv, b2r, zx, Wp, bpr)
    adj_recon = _tc_adj(z)
    return (adj_recon, feat_recon, pred, z)
